# Initial kernel scaffold; baseline (speedup 1.0000x reference)
import jax
import jax.numpy as jnp
from jax.experimental import pallas as pl


def kernel(edge_index, e1, rel, node_table, rel_emb, W_msg0, Wa0, Ua0, b0, gamma0, beta0, W_msg1, Wa1, Ua1, b1, gamma1, beta1):
    raise NotImplementedError("write your pallas kernel here")


# speedup vs baseline: 4.6571x; 4.6571x over previous
import functools

import jax
import jax.numpy as jnp
from jax import lax
from jax.experimental import pallas as pl
from jax.experimental.pallas import tpu as pltpu
from jax.experimental.pallas import tpu_sc as plsc

_N = 10000
_E = 320000
_D = 128
_B = 1024

_NC = 2
_NS = 16
_NW = _NC * _NS

_EPW = _E // _NW
_CHUNK = 80
_NCHUNK = _EPW // _CHUNK
_ROWS_PT = 624
_ROWS_TAIL = _N - _NS * _ROWS_PT

_BPW = _B // _NW

_BR = 1000
_NB = _N // _BR
_CB = 1000
_EPS = 1e-5

@functools.lru_cache(maxsize=None)
def _sc_mesh():
    return plsc.VectorSubcoreMesh(core_axis_name="c", subcore_axis_name="s",
                                  num_cores=_NC, num_subcores=_NS)


def _segsum_body(src_hbm, dst_hbm, h_hbm, zeros_hbm, out_hbm,
                 src_v, dst_v, rows_v, acc_sh, sem):
    cid = lax.axis_index("c")
    sid = lax.axis_index("s")
    wid = sid * _NC + cid
    rbase = pl.multiple_of(sid * _ROWS_PT, 8)
    pltpu.sync_copy(zeros_hbm.at[pl.ds(rbase, _ROWS_PT)],
                    acc_sh.at[pl.ds(rbase, _ROWS_PT)])

    @pl.when(sid == _NS - 1)
    def _zero_tail():
        tbase = _NS * _ROWS_PT
        pltpu.sync_copy(zeros_hbm.at[pl.ds(tbase, _ROWS_TAIL)],
                        acc_sh.at[pl.ds(tbase, _ROWS_TAIL)])

    plsc.subcore_barrier()

    ebase = wid * _EPW

    def body(i, carry):
        off = pl.multiple_of(ebase + i * _CHUNK, 8)
        pltpu.sync_copy(src_hbm.at[pl.ds(off, _CHUNK)], src_v)
        pltpu.sync_copy(dst_hbm.at[pl.ds(off, _CHUNK)], dst_v)
        pltpu.async_copy(h_hbm.at[src_v], rows_v, sem).wait()
        pltpu.sync_copy(rows_v, acc_sh.at[dst_v], add=True)
        return carry

    lax.fori_loop(0, _NCHUNK, body, 0)
    plsc.subcore_barrier()
    pltpu.sync_copy(acc_sh.at[pl.ds(rbase, _ROWS_PT)],
                    out_hbm.at[cid, pl.ds(rbase, _ROWS_PT)])

    @pl.when(sid == _NS - 1)
    def _write_tail():
        tbase = _NS * _ROWS_PT
        pltpu.sync_copy(acc_sh.at[pl.ds(tbase, _ROWS_TAIL)],
                        out_hbm.at[cid, pl.ds(tbase, _ROWS_TAIL)])


def _gather_body(e1_hbm, rel_hbm, h_hbm, remb_hbm, o1_hbm, o2_hbm,
                 idx_v, rows_v, sem):
    wid = lax.axis_index("s") * _NC + lax.axis_index("c")
    base = pl.multiple_of(wid * _BPW, 8)
    pltpu.sync_copy(e1_hbm.at[pl.ds(base, _BPW)], idx_v)
    pltpu.async_copy(h_hbm.at[idx_v], rows_v, sem).wait()
    pltpu.sync_copy(rows_v, o1_hbm.at[pl.ds(base, _BPW)])
    pltpu.sync_copy(rel_hbm.at[pl.ds(base, _BPW)], idx_v)
    pltpu.async_copy(remb_hbm.at[idx_v], rows_v, sem).wait()
    pltpu.sync_copy(rows_v, o2_hbm.at[pl.ds(base, _BPW)])


@functools.lru_cache(maxsize=None)
def _build_segsum_sc():
    return pl.kernel(
        _segsum_body,
        out_type=jax.ShapeDtypeStruct((_NC, _N, _D), jnp.float32),
        mesh=_sc_mesh(),
        scratch_types=[
            pltpu.VMEM((_CHUNK,), jnp.int32),
            pltpu.VMEM((_CHUNK,), jnp.int32),
            pltpu.VMEM((_CHUNK, _D), jnp.float32),
            pltpu.VMEM_SHARED((_N, _D), jnp.float32),
            pltpu.SemaphoreType.DMA,
        ],
    )


@functools.lru_cache(maxsize=None)
def _build_gather_sc():
    return pl.kernel(
        _gather_body,
        out_type=(jax.ShapeDtypeStruct((_B, _D), jnp.float32),
                  jax.ShapeDtypeStruct((_B, _D), jnp.float32)),
        mesh=_sc_mesh(),
        scratch_types=[
            pltpu.VMEM((_BPW,), jnp.int32),
            pltpu.VMEM((_BPW, _D), jnp.float32),
            pltpu.SemaphoreType.DMA,
        ],
    )


def _layer_body(h_ref, agg_ref, wm_ref, wa_ref, ua_ref, b_ref, gm_ref, bt_ref,
                out_ref, hpre_ref, stat_ref):
    p = pl.program_id(0)
    i = pl.program_id(1)

    @pl.when(p == 0)
    def _compute():
        h = h_ref[...]
        agg = agg_ref[0] + agg_ref[1]
        a = jnp.dot(agg, wm_ref[...], preferred_element_type=jnp.float32)
        ga = jnp.dot(a, wa_ref[...], preferred_element_type=jnp.float32) + b_ref[...]
        gh = jnp.dot(h, ua_ref[...], preferred_element_type=jnp.float32)
        z = jax.nn.sigmoid(ga[:, :_D] + gh[:, :_D])
        r = jax.nn.sigmoid(ga[:, _D:2 * _D] + gh[:, _D:2 * _D])
        h_til = jnp.tanh(ga[:, 2 * _D:] + r * gh[:, 2 * _D:])
        hp = (1.0 - z) * h + z * h_til
        hpre_ref[pl.ds(i * _BR, _BR), :] = hp

        @pl.when(i == 0)
        def _init():
            stat_ref[...] = jnp.zeros_like(stat_ref)

        stat_ref[0, :] += jnp.sum(hp, axis=0)
        stat_ref[1, :] += jnp.sum(hp * hp, axis=0)

    @pl.when(p == 1)
    def _normalize():
        mu = stat_ref[0, :] * (1.0 / _N)
        var = stat_ref[1, :] * (1.0 / _N) - mu * mu
        hp = hpre_ref[pl.ds(i * _BR, _BR), :]
        xn = (hp - mu) * lax.rsqrt(var + _EPS)
        out_ref[...] = jnp.tanh(gm_ref[...] * xn + bt_ref[...])


_layer_tc = pl.pallas_call(
    _layer_body,
    grid=(2, _NB),
    in_specs=[
        pl.BlockSpec((_BR, _D), lambda p, i: (i, 0)),
        pl.BlockSpec((_NC, _BR, _D), lambda p, i: (0, i, 0)),
        pl.BlockSpec((_D, _D), lambda p, i: (0, 0)),
        pl.BlockSpec((_D, 3 * _D), lambda p, i: (0, 0)),
        pl.BlockSpec((_D, 3 * _D), lambda p, i: (0, 0)),
        pl.BlockSpec((1, 3 * _D), lambda p, i: (0, 0)),
        pl.BlockSpec((1, _D), lambda p, i: (0, 0)),
        pl.BlockSpec((1, _D), lambda p, i: (0, 0)),
    ],
    out_specs=pl.BlockSpec((_BR, _D), lambda p, i: (i, 0)),
    out_shape=jax.ShapeDtypeStruct((_N, _D), jnp.float32),
    scratch_shapes=[
        pltpu.VMEM((_N, _D), jnp.float32),
        pltpu.VMEM((8, _D), jnp.float32),
    ],
)


def _distmult_body(e1_ref, r_ref, h_ref, out_ref):
    q = e1_ref[...] * r_ref[...]
    s = lax.dot_general(q, h_ref[...], (((1,), (1,)), ((), ())),
                        preferred_element_type=jnp.float32)
    out_ref[...] = jax.nn.sigmoid(s)


_QB = 128

_distmult_tc = pl.pallas_call(
    _distmult_body,
    grid=(_B // _QB,),
    in_specs=[
        pl.BlockSpec((_QB, _D), lambda j: (j, 0)),
        pl.BlockSpec((_QB, _D), lambda j: (j, 0)),
        pl.BlockSpec((_N, _D), lambda j: (0, 0)),
    ],
    out_specs=pl.BlockSpec((_QB, _N), lambda j: (j, 0)),
    out_shape=jax.ShapeDtypeStruct((_B, _N), jnp.float32),
)


def kernel(edge_index, e1, rel, node_table, rel_emb,
           W_msg0, Wa0, Ua0, b0, gamma0, beta0,
           W_msg1, Wa1, Ua1, b1, gamma1, beta1):
    src = edge_index[0]
    dst = edge_index[1]
    zeros = jnp.zeros((_N, _D), jnp.float32)
    b0r = b0.reshape(1, 3 * _D)
    b1r = b1.reshape(1, 3 * _D)
    g0r = gamma0.reshape(1, _D)
    g1r = gamma1.reshape(1, _D)
    bt0r = beta0.reshape(1, _D)
    bt1r = beta1.reshape(1, _D)

    segsum_sc = _build_segsum_sc()
    gather_sc = _build_gather_sc()
    agg0 = segsum_sc(src, dst, node_table, zeros)
    h1 = _layer_tc(node_table, agg0, W_msg0, Wa0, Ua0, b0r, g0r, bt0r)
    agg1 = segsum_sc(src, dst, h1, zeros)
    h2 = _layer_tc(h1, agg1, W_msg1, Wa1, Ua1, b1r, g1r, bt1r)
    e1_emb, r_emb = gather_sc(e1, rel, h2, rel_emb)
    return _distmult_tc(e1_emb, r_emb, h2)


# speedup vs baseline: 9.2821x; 1.9931x over previous
import functools

import jax
import jax.numpy as jnp
from jax import lax
from jax.experimental import pallas as pl
from jax.experimental.pallas import tpu as pltpu
from jax.experimental.pallas import tpu_sc as plsc

_N = 10000
_E = 320000
_D = 128
_B = 1024

_NC = 2
_NS = 16
_NW = _NC * _NS

_EPW = _E // _NW
_CHUNK = 80
_NCHUNK = _EPW // _CHUNK
_ROWS_PT = 624
_ROWS_TAIL = _N - _NS * _ROWS_PT

_BPW = _B // _NW

_BR = 1000
_NB = _N // _BR
_CB = 1000
_EPS = 1e-5

@functools.lru_cache(maxsize=None)
def _sc_mesh():
    return plsc.VectorSubcoreMesh(core_axis_name="c", subcore_axis_name="s",
                                  num_cores=_NC, num_subcores=_NS)


def _segsum_body(src_hbm, dst3_hbm, h_hbm, zeros_hbm, out_hbm,
                 src_all, dst_slab, rows_a, rows_b, acc_sh, sem_a, sem_b):
    cid = lax.axis_index("c")
    sid = lax.axis_index("s")
    wid = sid * _NC + cid
    rbase = pl.multiple_of(sid * _ROWS_PT, 8)
    ebase = pl.multiple_of(wid * _EPW, 8)

    cp_src = pltpu.async_copy(src_hbm.at[pl.ds(ebase, _EPW)], src_all, sem_a)
    cp_dst = pltpu.async_copy(dst3_hbm.at[wid], dst_slab, sem_b)
    pltpu.sync_copy(zeros_hbm.at[pl.ds(rbase, _ROWS_PT)],
                    acc_sh.at[pl.ds(rbase, _ROWS_PT)])

    @pl.when(sid == _NS - 1)
    def _zero_tail():
        tbase = _NS * _ROWS_PT
        pltpu.sync_copy(zeros_hbm.at[pl.ds(tbase, _ROWS_TAIL)],
                        acc_sh.at[pl.ds(tbase, _ROWS_TAIL)])

    cp_src.wait()
    cp_dst.wait()
    plsc.subcore_barrier()

    def gather_start(j, buf, sem):
        off = pl.multiple_of(j * _CHUNK, 8)
        pltpu.async_copy(h_hbm.at[src_all.at[pl.ds(off, _CHUNK)]], buf, sem)

    def gather_wait(buf, sem):
        pltpu.make_async_copy(h_hbm.at[src_all.at[pl.ds(0, _CHUNK)]],
                              buf, sem).wait()

    def scatter_add(j, buf):
        pltpu.sync_copy(buf, acc_sh.at[dst_slab.at[j]], add=True)

    gather_start(0, rows_a, sem_a)

    def body(p, carry):
        j = p * 2
        gather_start(j + 1, rows_b, sem_b)
        gather_wait(rows_a, sem_a)
        scatter_add(j, rows_a)
        gather_start(j + 2, rows_a, sem_a)
        gather_wait(rows_b, sem_b)
        scatter_add(j + 1, rows_b)
        return carry

    lax.fori_loop(0, (_NCHUNK - 1) // 2, body, 0)
    gather_wait(rows_a, sem_a)
    scatter_add(_NCHUNK - 1, rows_a)

    plsc.subcore_barrier()
    pltpu.sync_copy(acc_sh.at[pl.ds(rbase, _ROWS_PT)],
                    out_hbm.at[cid, pl.ds(rbase, _ROWS_PT)])

    @pl.when(sid == _NS - 1)
    def _write_tail():
        tbase = _NS * _ROWS_PT
        pltpu.sync_copy(acc_sh.at[pl.ds(tbase, _ROWS_TAIL)],
                        out_hbm.at[cid, pl.ds(tbase, _ROWS_TAIL)])


def _gather_body(e1_hbm, rel_hbm, h_hbm, remb_hbm, o1_hbm, o2_hbm,
                 idx_v, rows_v, sem):
    wid = lax.axis_index("s") * _NC + lax.axis_index("c")
    base = pl.multiple_of(wid * _BPW, 8)
    pltpu.sync_copy(e1_hbm.at[pl.ds(base, _BPW)], idx_v)
    pltpu.async_copy(h_hbm.at[idx_v], rows_v, sem).wait()
    pltpu.sync_copy(rows_v, o1_hbm.at[pl.ds(base, _BPW)])
    pltpu.sync_copy(rel_hbm.at[pl.ds(base, _BPW)], idx_v)
    pltpu.async_copy(remb_hbm.at[idx_v], rows_v, sem).wait()
    pltpu.sync_copy(rows_v, o2_hbm.at[pl.ds(base, _BPW)])


@functools.lru_cache(maxsize=None)
def _build_segsum_sc():
    return pl.kernel(
        _segsum_body,
        out_type=jax.ShapeDtypeStruct((_NC, _N, _D), jnp.float32),
        mesh=_sc_mesh(),
        scratch_types=[
            pltpu.VMEM((_EPW,), jnp.int32),
            pltpu.VMEM((_NCHUNK, _CHUNK), jnp.int32),
            pltpu.VMEM((_CHUNK, _D), jnp.float32),
            pltpu.VMEM((_CHUNK, _D), jnp.float32),
            pltpu.VMEM_SHARED((_N, _D), jnp.float32),
            pltpu.SemaphoreType.DMA,
            pltpu.SemaphoreType.DMA,
        ],
    )


@functools.lru_cache(maxsize=None)
def _build_gather_sc():
    return pl.kernel(
        _gather_body,
        out_type=(jax.ShapeDtypeStruct((_B, _D), jnp.float32),
                  jax.ShapeDtypeStruct((_B, _D), jnp.float32)),
        mesh=_sc_mesh(),
        scratch_types=[
            pltpu.VMEM((_BPW,), jnp.int32),
            pltpu.VMEM((_BPW, _D), jnp.float32),
            pltpu.SemaphoreType.DMA,
        ],
    )


def _layer_body(h_ref, agg_ref, wm_ref, wa_ref, ua_ref, b_ref, gm_ref, bt_ref,
                out_ref, hpre_ref, stat_ref):
    p = pl.program_id(0)
    i = pl.program_id(1)

    @pl.when(p == 0)
    def _compute():
        h = h_ref[...]
        agg = agg_ref[0] + agg_ref[1]
        a = jnp.dot(agg, wm_ref[...], preferred_element_type=jnp.float32)
        ga = jnp.dot(a, wa_ref[...], preferred_element_type=jnp.float32) + b_ref[...]
        gh = jnp.dot(h, ua_ref[...], preferred_element_type=jnp.float32)
        z = jax.nn.sigmoid(ga[:, :_D] + gh[:, :_D])
        r = jax.nn.sigmoid(ga[:, _D:2 * _D] + gh[:, _D:2 * _D])
        h_til = jnp.tanh(ga[:, 2 * _D:] + r * gh[:, 2 * _D:])
        hp = (1.0 - z) * h + z * h_til
        hpre_ref[pl.ds(i * _BR, _BR), :] = hp

        @pl.when(i == 0)
        def _init():
            stat_ref[...] = jnp.zeros_like(stat_ref)

        stat_ref[0, :] += jnp.sum(hp, axis=0)
        stat_ref[1, :] += jnp.sum(hp * hp, axis=0)

    @pl.when(p == 1)
    def _normalize():
        mu = stat_ref[0, :] * (1.0 / _N)
        var = stat_ref[1, :] * (1.0 / _N) - mu * mu
        hp = hpre_ref[pl.ds(i * _BR, _BR), :]
        xn = (hp - mu) * lax.rsqrt(var + _EPS)
        out_ref[...] = jnp.tanh(gm_ref[...] * xn + bt_ref[...])


_layer_tc = pl.pallas_call(
    _layer_body,
    grid=(2, _NB),
    in_specs=[
        pl.BlockSpec((_BR, _D), lambda p, i: (i, 0)),
        pl.BlockSpec((_NC, _BR, _D), lambda p, i: (0, i, 0)),
        pl.BlockSpec((_D, _D), lambda p, i: (0, 0)),
        pl.BlockSpec((_D, 3 * _D), lambda p, i: (0, 0)),
        pl.BlockSpec((_D, 3 * _D), lambda p, i: (0, 0)),
        pl.BlockSpec((1, 3 * _D), lambda p, i: (0, 0)),
        pl.BlockSpec((1, _D), lambda p, i: (0, 0)),
        pl.BlockSpec((1, _D), lambda p, i: (0, 0)),
    ],
    out_specs=pl.BlockSpec((_BR, _D), lambda p, i: (i, 0)),
    out_shape=jax.ShapeDtypeStruct((_N, _D), jnp.float32),
    scratch_shapes=[
        pltpu.VMEM((_N, _D), jnp.float32),
        pltpu.VMEM((8, _D), jnp.float32),
    ],
)


def _distmult_body(e1_ref, r_ref, h_ref, out_ref):
    q = e1_ref[...] * r_ref[...]
    s = lax.dot_general(q, h_ref[...], (((1,), (1,)), ((), ())),
                        preferred_element_type=jnp.float32)
    out_ref[...] = jax.nn.sigmoid(s)


_QB = 128

_distmult_tc = pl.pallas_call(
    _distmult_body,
    grid=(_B // _QB,),
    in_specs=[
        pl.BlockSpec((_QB, _D), lambda j: (j, 0)),
        pl.BlockSpec((_QB, _D), lambda j: (j, 0)),
        pl.BlockSpec((_N, _D), lambda j: (0, 0)),
    ],
    out_specs=pl.BlockSpec((_QB, _N), lambda j: (j, 0)),
    out_shape=jax.ShapeDtypeStruct((_B, _N), jnp.float32),
)


def kernel(edge_index, e1, rel, node_table, rel_emb,
           W_msg0, Wa0, Ua0, b0, gamma0, beta0,
           W_msg1, Wa1, Ua1, b1, gamma1, beta1):
    src = edge_index[0]
    dst = edge_index[1]
    zeros = jnp.zeros((_N, _D), jnp.float32)
    b0r = b0.reshape(1, 3 * _D)
    b1r = b1.reshape(1, 3 * _D)
    g0r = gamma0.reshape(1, _D)
    g1r = gamma1.reshape(1, _D)
    bt0r = beta0.reshape(1, _D)
    bt1r = beta1.reshape(1, _D)

    segsum_sc = _build_segsum_sc()
    gather_sc = _build_gather_sc()
    dst3 = dst.reshape(_NW, _NCHUNK, _CHUNK)
    agg0 = segsum_sc(src, dst3, node_table, zeros)
    h1 = _layer_tc(node_table, agg0, W_msg0, Wa0, Ua0, b0r, g0r, bt0r)
    agg1 = segsum_sc(src, dst3, h1, zeros)
    h2 = _layer_tc(h1, agg1, W_msg1, Wa1, Ua1, b1r, g1r, bt1r)
    e1_emb, r_emb = gather_sc(e1, rel, h2, rel_emb)
    return _distmult_tc(e1_emb, r_emb, h2)


# speedup vs baseline: 10.7832x; 1.1617x over previous
import functools

import jax
import jax.numpy as jnp
from jax import lax
from jax.experimental import pallas as pl
from jax.experimental.pallas import tpu as pltpu
from jax.experimental.pallas import tpu_sc as plsc

_N = 10000
_E = 320000
_D = 128
_B = 1024

_NC = 2
_NS = 16
_NW = _NC * _NS

_EPW = _E // _NW
_CHUNK = 80
_NCHUNK = _EPW // _CHUNK
_NBUF = 4
_NPIECE = 5
_EPH = _EPW // _NPIECE
_CHPH = _NCHUNK // _NPIECE
_ROWS_PT = 624
_ROWS_TAIL = _N - _NS * _ROWS_PT

_BPW = _B // _NW

_BR = 1000
_NB = _N // _BR
_CB = 1000
_EPS = 1e-5

@functools.lru_cache(maxsize=None)
def _sc_mesh():
    return plsc.VectorSubcoreMesh(core_axis_name="c", subcore_axis_name="s",
                                  num_cores=_NC, num_subcores=_NS)


def _segsum_body(src_hbm, dst_hbm, h_hbm, zeros_hbm, out_hbm,
                 src_half, dst_half, *rest):
    rows = rest[:_NBUF]
    acc_sh = rest[_NBUF]
    sems = rest[_NBUF + 1:]
    gsems = sems[:_NBUF]
    ssems = sems[_NBUF:2 * _NBUF]
    isem_a, isem_b = sems[2 * _NBUF:]
    cid = lax.axis_index("c")
    sid = lax.axis_index("s")
    wid = sid * _NC + cid
    rbase = pl.multiple_of(sid * _ROWS_PT, 8)

    pltpu.sync_copy(zeros_hbm.at[pl.ds(rbase, _ROWS_PT)],
                    acc_sh.at[pl.ds(rbase, _ROWS_PT)])

    @pl.when(sid == _NS - 1)
    def _zero_tail():
        tbase = _NS * _ROWS_PT
        pltpu.sync_copy(zeros_hbm.at[pl.ds(tbase, _ROWS_TAIL)],
                        acc_sh.at[pl.ds(tbase, _ROWS_TAIL)])

    plsc.subcore_barrier()

    def gather_start(j, buf, sem):
        off = pl.multiple_of(j * _CHUNK, 8)
        pltpu.async_copy(h_hbm.at[src_half.at[pl.ds(off, _CHUNK)]], buf, sem)

    def gather_wait(buf, sem):
        pltpu.make_async_copy(h_hbm.at[src_half.at[pl.ds(0, _CHUNK)]],
                              buf, sem).wait()

    def scat_start(j, buf, sem):
        off = pl.multiple_of(j * _CHUNK, 8)
        pltpu.sync_copy(buf, acc_sh.at[dst_half.at[pl.ds(off, _CHUNK)]],
                        add=True)

    def scat_wait(buf, sem):
        pass

    for half in range(_NPIECE):
        ebase = pl.multiple_of(wid * _EPW + half * _EPH, 8)
        cp_src = pltpu.async_copy(src_hbm.at[pl.ds(ebase, _EPH)], src_half,
                                  isem_a)
        cp_dst = pltpu.async_copy(dst_hbm.at[pl.ds(ebase, _EPH)], dst_half,
                                  isem_b)
        cp_src.wait()
        cp_dst.wait()

        for k in range(_NBUF):
            gather_start(k, rows[k], gsems[k])

        def body(p, carry):
            j = p * _NBUF
            for k in range(_NBUF):
                gather_wait(rows[k], gsems[k])
                scat_start(j + k, rows[k], ssems[k])
                nj = j + _NBUF + k

                @pl.when(nj < _CHPH)
                def _next():
                    gather_start(nj, rows[k], gsems[k])

            return carry

        lax.fori_loop(0, _CHPH // _NBUF, body, 0)
        jbase = (_CHPH // _NBUF) * _NBUF
        for k in range(_CHPH % _NBUF):
            gather_wait(rows[k], gsems[k])
            scat_start(jbase + k, rows[k], ssems[k])

    plsc.subcore_barrier()
    pltpu.sync_copy(acc_sh.at[pl.ds(rbase, _ROWS_PT)],
                    out_hbm.at[cid, pl.ds(rbase, _ROWS_PT)])

    @pl.when(sid == _NS - 1)
    def _write_tail():
        tbase = _NS * _ROWS_PT
        pltpu.sync_copy(acc_sh.at[pl.ds(tbase, _ROWS_TAIL)],
                        out_hbm.at[cid, pl.ds(tbase, _ROWS_TAIL)])


def _gather_body(e1_hbm, rel_hbm, h_hbm, remb_hbm, o1_hbm, o2_hbm,
                 idx_v, rows_v, sem):
    wid = lax.axis_index("s") * _NC + lax.axis_index("c")
    base = pl.multiple_of(wid * _BPW, 8)
    pltpu.sync_copy(e1_hbm.at[pl.ds(base, _BPW)], idx_v)
    pltpu.async_copy(h_hbm.at[idx_v], rows_v, sem).wait()
    pltpu.sync_copy(rows_v, o1_hbm.at[pl.ds(base, _BPW)])
    pltpu.sync_copy(rel_hbm.at[pl.ds(base, _BPW)], idx_v)
    pltpu.async_copy(remb_hbm.at[idx_v], rows_v, sem).wait()
    pltpu.sync_copy(rows_v, o2_hbm.at[pl.ds(base, _BPW)])


@functools.lru_cache(maxsize=None)
def _build_segsum_sc():
    return pl.kernel(
        _segsum_body,
        out_type=jax.ShapeDtypeStruct((_NC, _N, _D), jnp.float32),
        mesh=_sc_mesh(),
        scratch_types=(
            [pltpu.VMEM((_EPH,), jnp.int32),
             pltpu.VMEM((_EPH,), jnp.int32)]
            + [pltpu.VMEM((_CHUNK, _D), jnp.float32)] * _NBUF
            + [pltpu.VMEM_SHARED((_N, _D), jnp.float32)]
            + [pltpu.SemaphoreType.DMA] * (2 * _NBUF + 2)
        ),
    )


@functools.lru_cache(maxsize=None)
def _build_gather_sc():
    return pl.kernel(
        _gather_body,
        out_type=(jax.ShapeDtypeStruct((_B, _D), jnp.float32),
                  jax.ShapeDtypeStruct((_B, _D), jnp.float32)),
        mesh=_sc_mesh(),
        scratch_types=[
            pltpu.VMEM((_BPW,), jnp.int32),
            pltpu.VMEM((_BPW, _D), jnp.float32),
            pltpu.SemaphoreType.DMA,
        ],
    )


def _gh_body(h_ref, ua_ref, out_ref):
    out_ref[...] = jnp.dot(h_ref[...], ua_ref[...],
                           preferred_element_type=jnp.float32)


_gh_tc = pl.pallas_call(
    _gh_body,
    grid=(_NB,),
    in_specs=[
        pl.BlockSpec((_BR, _D), lambda i: (i, 0)),
        pl.BlockSpec((_D, 3 * _D), lambda i: (0, 0)),
    ],
    out_specs=pl.BlockSpec((_BR, 3 * _D), lambda i: (i, 0)),
    out_shape=jax.ShapeDtypeStruct((_N, 3 * _D), jnp.float32),
)


def _layer_body(h_ref, agg_ref, wm_ref, wa_ref, gh_ref, b_ref, gm_ref, bt_ref,
                out_ref, hpre_ref, stat_ref):
    p = pl.program_id(0)
    i = pl.program_id(1)

    @pl.when(p == 0)
    def _compute():
        h = h_ref[...]
        agg = agg_ref[0] + agg_ref[1]
        a = jnp.dot(agg, wm_ref[...], preferred_element_type=jnp.float32)
        ga = jnp.dot(a, wa_ref[...], preferred_element_type=jnp.float32) + b_ref[...]
        gh = gh_ref[...]
        z = jax.nn.sigmoid(ga[:, :_D] + gh[:, :_D])
        r = jax.nn.sigmoid(ga[:, _D:2 * _D] + gh[:, _D:2 * _D])
        h_til = jnp.tanh(ga[:, 2 * _D:] + r * gh[:, 2 * _D:])
        hp = (1.0 - z) * h + z * h_til
        hpre_ref[pl.ds(i * _BR, _BR), :] = hp

        @pl.when(i == 0)
        def _init():
            stat_ref[...] = jnp.zeros_like(stat_ref)

        stat_ref[0, :] += jnp.sum(hp, axis=0)
        stat_ref[1, :] += jnp.sum(hp * hp, axis=0)

    @pl.when(p == 1)
    def _normalize():
        mu = stat_ref[0, :] * (1.0 / _N)
        var = stat_ref[1, :] * (1.0 / _N) - mu * mu
        hp = hpre_ref[pl.ds(i * _BR, _BR), :]
        xn = (hp - mu) * lax.rsqrt(var + _EPS)
        out_ref[...] = jnp.tanh(gm_ref[...] * xn + bt_ref[...])


_layer_tc = pl.pallas_call(
    _layer_body,
    grid=(2, _NB),
    in_specs=[
        pl.BlockSpec((_BR, _D), lambda p, i: (i, 0)),
        pl.BlockSpec((_NC, _BR, _D), lambda p, i: (0, i, 0)),
        pl.BlockSpec((_D, _D), lambda p, i: (0, 0)),
        pl.BlockSpec((_D, 3 * _D), lambda p, i: (0, 0)),
        pl.BlockSpec((_BR, 3 * _D), lambda p, i: (i, 0)),
        pl.BlockSpec((1, 3 * _D), lambda p, i: (0, 0)),
        pl.BlockSpec((1, _D), lambda p, i: (0, 0)),
        pl.BlockSpec((1, _D), lambda p, i: (0, 0)),
    ],
    out_specs=pl.BlockSpec((_BR, _D), lambda p, i: (i, 0)),
    out_shape=jax.ShapeDtypeStruct((_N, _D), jnp.float32),
    scratch_shapes=[
        pltpu.VMEM((_N, _D), jnp.float32),
        pltpu.VMEM((8, _D), jnp.float32),
    ],
)


def _distmult_body(e1_ref, r_ref, h_ref, out_ref):
    q = e1_ref[...] * r_ref[...]
    s = lax.dot_general(h_ref[...], q, (((1,), (1,)), ((), ())),
                        preferred_element_type=jnp.float32)
    out_ref[...] = jax.nn.sigmoid(s)


_EB = 1000

_distmult_tc = pl.pallas_call(
    _distmult_body,
    grid=(_N // _EB,),
    in_specs=[
        pl.BlockSpec((_B, _D), lambda j: (0, 0)),
        pl.BlockSpec((_B, _D), lambda j: (0, 0)),
        pl.BlockSpec((_EB, _D), lambda j: (j, 0)),
    ],
    out_specs=pl.BlockSpec((_EB, _B), lambda j: (j, 0)),
    out_shape=jax.ShapeDtypeStruct((_N, _B), jnp.float32),
)


def kernel(edge_index, e1, rel, node_table, rel_emb,
           W_msg0, Wa0, Ua0, b0, gamma0, beta0,
           W_msg1, Wa1, Ua1, b1, gamma1, beta1):
    zeros = jnp.zeros((_N, _D), jnp.float32)
    b0r = b0.reshape(1, 3 * _D)
    b1r = b1.reshape(1, 3 * _D)
    g0r = gamma0.reshape(1, _D)
    g1r = gamma1.reshape(1, _D)
    bt0r = beta0.reshape(1, _D)
    bt1r = beta1.reshape(1, _D)

    segsum_sc = _build_segsum_sc()
    gather_sc = _build_gather_sc()
    src = edge_index[0]
    dst = edge_index[1]
    agg0 = segsum_sc(src, dst, node_table, zeros)
    gh0 = _gh_tc(node_table, Ua0)
    h1 = _layer_tc(node_table, agg0, W_msg0, Wa0, gh0, b0r, g0r, bt0r)
    agg1 = segsum_sc(src, dst, h1, zeros)
    gh1 = _gh_tc(h1, Ua1)
    h2 = _layer_tc(h1, agg1, W_msg1, Wa1, gh1, b1r, g1r, bt1r)
    e1_emb, r_emb = gather_sc(e1, rel, h2, rel_emb)
    return _distmult_tc(e1_emb, r_emb, h2).T


# speedup vs baseline: 11.4350x; 1.0604x over previous
import functools

import jax
import jax.numpy as jnp
from jax import lax
from jax.experimental import pallas as pl
from jax.experimental.pallas import tpu as pltpu
from jax.experimental.pallas import tpu_sc as plsc

_N = 10000
_E = 320000
_D = 128
_B = 1024

_NC = 2
_NS = 16
_NW = _NC * _NS

_EPW = _E // _NW
_CHUNK = 80
_NCHUNK = _EPW // _CHUNK
_NBUF = 4
_NPIECE = 5
_EPH = _EPW // _NPIECE
_CHPH = _NCHUNK // _NPIECE
_ROWS_PT = 624
_ROWS_TAIL = _N - _NS * _ROWS_PT

_BPW = _B // _NW

_BR = 1000
_NB = _N // _BR
_CB = 1000
_EPS = 1e-5

@functools.lru_cache(maxsize=None)
def _sc_mesh():
    return plsc.VectorSubcoreMesh(core_axis_name="c", subcore_axis_name="s",
                                  num_cores=_NC, num_subcores=_NS)


def _segsum_body(src_hbm, dst_hbm, h_hbm, zeros_hbm, out_hbm,
                 src_sl0, src_sl1, dst_sl0, dst_sl1, *rest):
    src_sl = (src_sl0, src_sl1)
    dst_sl = (dst_sl0, dst_sl1)
    rows = rest[:_NBUF]
    acc_sh = rest[_NBUF]
    sems = rest[_NBUF + 1:]
    gsems = sems[:_NBUF]
    isems = sems[_NBUF:_NBUF + 4]
    cid = lax.axis_index("c")
    sid = lax.axis_index("s")
    wid = sid * _NC + cid
    rbase = pl.multiple_of(sid * _ROWS_PT, 8)

    def slab_load(h, par):
        eb = pl.multiple_of(wid * _EPW + h * _EPH, 8)
        pltpu.async_copy(src_hbm.at[pl.ds(eb, _EPH)], src_sl[par],
                         isems[2 * par])
        pltpu.async_copy(dst_hbm.at[pl.ds(eb, _EPH)], dst_sl[par],
                         isems[2 * par + 1])

    def slab_wait(par):
        pltpu.make_async_copy(src_hbm.at[pl.ds(0, _EPH)], src_sl[par],
                              isems[2 * par]).wait()
        pltpu.make_async_copy(dst_hbm.at[pl.ds(0, _EPH)], dst_sl[par],
                              isems[2 * par + 1]).wait()

    slab_load(0, 0)
    pltpu.sync_copy(zeros_hbm.at[pl.ds(rbase, _ROWS_PT)],
                    acc_sh.at[pl.ds(rbase, _ROWS_PT)])

    @pl.when(sid == _NS - 1)
    def _zero_tail():
        tbase = _NS * _ROWS_PT
        pltpu.sync_copy(zeros_hbm.at[pl.ds(tbase, _ROWS_TAIL)],
                        acc_sh.at[pl.ds(tbase, _ROWS_TAIL)])

    plsc.subcore_barrier()

    def gather_start(j, buf, sem, ssl):
        off = pl.multiple_of(j * _CHUNK, 8)
        pltpu.async_copy(h_hbm.at[ssl.at[pl.ds(off, _CHUNK)]], buf, sem)

    def gather_wait(buf, sem, ssl):
        pltpu.make_async_copy(h_hbm.at[ssl.at[pl.ds(0, _CHUNK)]],
                              buf, sem).wait()

    def scat_start(j, buf, dsl):
        off = pl.multiple_of(j * _CHUNK, 8)
        pltpu.sync_copy(buf, acc_sh.at[dsl.at[pl.ds(off, _CHUNK)]],
                        add=True)

    for half in range(_NPIECE):
        par = half % 2
        ssl = src_sl[par]
        dsl = dst_sl[par]
        slab_wait(par)
        if half + 1 < _NPIECE:
            slab_load(half + 1, (half + 1) % 2)

        for k in range(_NBUF):
            gather_start(k, rows[k], gsems[k], ssl)

        def body(p, carry):
            j = p * _NBUF
            for k in range(_NBUF):
                gather_wait(rows[k], gsems[k], ssl)
                scat_start(j + k, rows[k], dsl)
                nj = j + _NBUF + k

                @pl.when(nj < _CHPH)
                def _next():
                    gather_start(nj, rows[k], gsems[k], ssl)

            return carry

        lax.fori_loop(0, _CHPH // _NBUF, body, 0)
        jbase = (_CHPH // _NBUF) * _NBUF
        for k in range(_CHPH % _NBUF):
            gather_wait(rows[k], gsems[k], ssl)
            scat_start(jbase + k, rows[k], dsl)

    plsc.subcore_barrier()
    pltpu.sync_copy(acc_sh.at[pl.ds(rbase, _ROWS_PT)],
                    out_hbm.at[cid, pl.ds(rbase, _ROWS_PT)])

    @pl.when(sid == _NS - 1)
    def _write_tail():
        tbase = _NS * _ROWS_PT
        pltpu.sync_copy(acc_sh.at[pl.ds(tbase, _ROWS_TAIL)],
                        out_hbm.at[cid, pl.ds(tbase, _ROWS_TAIL)])


def _gather_body(e1_hbm, rel_hbm, h_hbm, remb_hbm, o1_hbm, o2_hbm,
                 idx_v, rows_v, sem):
    wid = lax.axis_index("s") * _NC + lax.axis_index("c")
    base = pl.multiple_of(wid * _BPW, 8)
    pltpu.sync_copy(e1_hbm.at[pl.ds(base, _BPW)], idx_v)
    pltpu.async_copy(h_hbm.at[idx_v], rows_v, sem).wait()
    pltpu.sync_copy(rows_v, o1_hbm.at[pl.ds(base, _BPW)])
    pltpu.sync_copy(rel_hbm.at[pl.ds(base, _BPW)], idx_v)
    pltpu.async_copy(remb_hbm.at[idx_v], rows_v, sem).wait()
    pltpu.sync_copy(rows_v, o2_hbm.at[pl.ds(base, _BPW)])


@functools.lru_cache(maxsize=None)
def _build_segsum_sc():
    return pl.kernel(
        _segsum_body,
        out_type=jax.ShapeDtypeStruct((_NC, _N, _D), jnp.float32),
        mesh=_sc_mesh(),
        scratch_types=(
            [pltpu.VMEM((_EPH,), jnp.int32)] * 2
            + [pltpu.VMEM((_EPH,), jnp.int32)] * 2
            + [pltpu.VMEM((_CHUNK, _D), jnp.float32)] * _NBUF
            + [pltpu.VMEM_SHARED((_N, _D), jnp.float32)]
            + [pltpu.SemaphoreType.DMA] * (_NBUF + 4)
        ),
    )


@functools.lru_cache(maxsize=None)
def _build_gather_sc():
    return pl.kernel(
        _gather_body,
        out_type=(jax.ShapeDtypeStruct((_B, _D), jnp.float32),
                  jax.ShapeDtypeStruct((_B, _D), jnp.float32)),
        mesh=_sc_mesh(),
        scratch_types=[
            pltpu.VMEM((_BPW,), jnp.int32),
            pltpu.VMEM((_BPW, _D), jnp.float32),
            pltpu.SemaphoreType.DMA,
        ],
    )


def _layer_body(h_ref, agg_ref, wm_ref, wa_ref, ua_ref, b_ref, gm_ref, bt_ref,
                out_ref, hpre_ref, stat_ref):
    p = pl.program_id(0)
    i = pl.program_id(1)

    @pl.when(p == 0)
    def _compute():
        h = h_ref[...]
        agg = agg_ref[0] + agg_ref[1]
        a = jnp.dot(agg, wm_ref[...], preferred_element_type=jnp.float32)
        ga = jnp.dot(a, wa_ref[...], preferred_element_type=jnp.float32) + b_ref[...]
        gh = jnp.dot(h, ua_ref[...], preferred_element_type=jnp.float32)
        z = jax.nn.sigmoid(ga[:, :_D] + gh[:, :_D])
        r = jax.nn.sigmoid(ga[:, _D:2 * _D] + gh[:, _D:2 * _D])
        h_til = jnp.tanh(ga[:, 2 * _D:] + r * gh[:, 2 * _D:])
        hp = (1.0 - z) * h + z * h_til
        hpre_ref[pl.ds(i * _BR, _BR), :] = hp

        @pl.when(i == 0)
        def _init():
            stat_ref[...] = jnp.zeros_like(stat_ref)

        stat_ref[0, :] += jnp.sum(hp, axis=0)
        stat_ref[1, :] += jnp.sum(hp * hp, axis=0)

    @pl.when(p == 1)
    def _normalize():
        mu = stat_ref[0, :] * (1.0 / _N)
        var = stat_ref[1, :] * (1.0 / _N) - mu * mu
        hp = hpre_ref[pl.ds(i * _BR, _BR), :]
        xn = (hp - mu) * lax.rsqrt(var + _EPS)
        out_ref[...] = jnp.tanh(gm_ref[...] * xn + bt_ref[...])


_layer_tc = pl.pallas_call(
    _layer_body,
    grid=(2, _NB),
    in_specs=[
        pl.BlockSpec((_BR, _D), lambda p, i: (i, 0)),
        pl.BlockSpec((_NC, _BR, _D), lambda p, i: (0, i, 0)),
        pl.BlockSpec((_D, _D), lambda p, i: (0, 0)),
        pl.BlockSpec((_D, 3 * _D), lambda p, i: (0, 0)),
        pl.BlockSpec((_D, 3 * _D), lambda p, i: (0, 0)),
        pl.BlockSpec((1, 3 * _D), lambda p, i: (0, 0)),
        pl.BlockSpec((1, _D), lambda p, i: (0, 0)),
        pl.BlockSpec((1, _D), lambda p, i: (0, 0)),
    ],
    out_specs=pl.BlockSpec((_BR, _D), lambda p, i: (i, 0)),
    out_shape=jax.ShapeDtypeStruct((_N, _D), jnp.float32),
    scratch_shapes=[
        pltpu.VMEM((_N, _D), jnp.float32),
        pltpu.VMEM((8, _D), jnp.float32),
    ],
)


def _distmult_body(e1_ref, r_ref, h_ref, out_ref):
    q = (e1_ref[...] * r_ref[...]).astype(jnp.bfloat16)
    s = lax.dot_general(h_ref[...].astype(jnp.bfloat16), q,
                        (((1,), (1,)), ((), ())),
                        preferred_element_type=jnp.float32)
    out_ref[...] = jax.nn.sigmoid(s)


_EB = 1000

_distmult_tc = pl.pallas_call(
    _distmult_body,
    grid=(_N // _EB,),
    in_specs=[
        pl.BlockSpec((_B, _D), lambda j: (0, 0)),
        pl.BlockSpec((_B, _D), lambda j: (0, 0)),
        pl.BlockSpec((_EB, _D), lambda j: (j, 0)),
    ],
    out_specs=pl.BlockSpec((_EB, _B), lambda j: (j, 0)),
    out_shape=jax.ShapeDtypeStruct((_N, _B), jnp.float32),
)


def kernel(edge_index, e1, rel, node_table, rel_emb,
           W_msg0, Wa0, Ua0, b0, gamma0, beta0,
           W_msg1, Wa1, Ua1, b1, gamma1, beta1):
    zeros = jnp.zeros((_N, _D), jnp.float32)
    b0r = b0.reshape(1, 3 * _D)
    b1r = b1.reshape(1, 3 * _D)
    g0r = gamma0.reshape(1, _D)
    g1r = gamma1.reshape(1, _D)
    bt0r = beta0.reshape(1, _D)
    bt1r = beta1.reshape(1, _D)

    segsum_sc = _build_segsum_sc()
    gather_sc = _build_gather_sc()
    src = edge_index[0]
    dst = edge_index[1]
    agg0 = segsum_sc(src, dst, node_table, zeros)
    h1 = _layer_tc(node_table, agg0, W_msg0, Wa0, Ua0, b0r, g0r, bt0r)
    agg1 = segsum_sc(src, dst, h1, zeros)
    h2 = _layer_tc(h1, agg1, W_msg1, Wa1, Ua1, b1r, g1r, bt1r)
    e1_emb, r_emb = gather_sc(e1, rel, h2, rel_emb)
    return _distmult_tc(e1_emb, r_emb, h2).T


# speedup vs baseline: 11.4482x; 1.0012x over previous
import functools

import jax
import jax.numpy as jnp
from jax import lax
from jax.experimental import pallas as pl
from jax.experimental.pallas import tpu as pltpu
from jax.experimental.pallas import tpu_sc as plsc

_N = 10000
_E = 320000
_D = 128
_B = 1024

_NC = 2
_NS = 16
_NW = _NC * _NS

_EPW = _E // _NW
_CHUNK = 80
_NCHUNK = _EPW // _CHUNK
_NBUF = 4
_NPIECE = 5
_EPH = _EPW // _NPIECE
_CHPH = _NCHUNK // _NPIECE
_ROWS_PT = 624
_ROWS_TAIL = _N - _NS * _ROWS_PT

_BPW = _B // _NW

_BR = 1000
_NB = _N // _BR
_CB = 1000
_EPS = 1e-5

@functools.lru_cache(maxsize=None)
def _sc_mesh():
    return plsc.VectorSubcoreMesh(core_axis_name="c", subcore_axis_name="s",
                                  num_cores=_NC, num_subcores=_NS)


def _segsum_body(src_hbm, dst_hbm, h_hbm, zeros_hbm, out_hbm,
                 src_sl0, src_sl1, dst_sl0, dst_sl1, *rest):
    src_sl = (src_sl0, src_sl1)
    dst_sl = (dst_sl0, dst_sl1)
    rows = rest[:_NBUF]
    acc_sh = rest[_NBUF]
    sems = rest[_NBUF + 1:]
    gsems = sems[:_NBUF]
    isems = sems[_NBUF:_NBUF + 4]
    cid = lax.axis_index("c")
    sid = lax.axis_index("s")
    wid = sid * _NC + cid
    rbase = pl.multiple_of(sid * _ROWS_PT, 8)

    def slab_load(h, par):
        eb = pl.multiple_of(wid * _EPW + h * _EPH, 8)
        pltpu.async_copy(src_hbm.at[pl.ds(eb, _EPH)], src_sl[par],
                         isems[2 * par])
        pltpu.async_copy(dst_hbm.at[pl.ds(eb, _EPH)], dst_sl[par],
                         isems[2 * par + 1])

    def slab_wait(par):
        pltpu.make_async_copy(src_hbm.at[pl.ds(0, _EPH)], src_sl[par],
                              isems[2 * par]).wait()
        pltpu.make_async_copy(dst_hbm.at[pl.ds(0, _EPH)], dst_sl[par],
                              isems[2 * par + 1]).wait()

    slab_load(0, 0)
    pltpu.sync_copy(zeros_hbm.at[pl.ds(rbase, _ROWS_PT)],
                    acc_sh.at[pl.ds(rbase, _ROWS_PT)])

    @pl.when(sid == _NS - 1)
    def _zero_tail():
        tbase = _NS * _ROWS_PT
        pltpu.sync_copy(zeros_hbm.at[pl.ds(tbase, _ROWS_TAIL)],
                        acc_sh.at[pl.ds(tbase, _ROWS_TAIL)])

    plsc.subcore_barrier()

    def gather_start(j, buf, sem, ssl):
        off = pl.multiple_of(j * _CHUNK, 8)
        pltpu.async_copy(h_hbm.at[ssl.at[pl.ds(off, _CHUNK)]], buf, sem)

    def gather_wait(buf, sem, ssl):
        pltpu.make_async_copy(h_hbm.at[ssl.at[pl.ds(0, _CHUNK)]],
                              buf, sem).wait()

    def scat_start(j, buf, dsl):
        off = pl.multiple_of(j * _CHUNK, 8)
        pltpu.sync_copy(buf, acc_sh.at[dsl.at[pl.ds(off, _CHUNK)]],
                        add=True)

    for half in range(_NPIECE):
        par = half % 2
        ssl = src_sl[par]
        dsl = dst_sl[par]
        slab_wait(par)
        if half + 1 < _NPIECE:
            slab_load(half + 1, (half + 1) % 2)

        for k in range(_NBUF):
            gather_start(k, rows[k], gsems[k], ssl)

        def body(p, carry):
            j = p * _NBUF
            for k in range(_NBUF):
                gather_wait(rows[k], gsems[k], ssl)
                scat_start(j + k, rows[k], dsl)
                nj = j + _NBUF + k

                @pl.when(nj < _CHPH)
                def _next():
                    gather_start(nj, rows[k], gsems[k], ssl)

            return carry

        lax.fori_loop(0, _CHPH // _NBUF, body, 0)
        jbase = (_CHPH // _NBUF) * _NBUF
        for k in range(_CHPH % _NBUF):
            gather_wait(rows[k], gsems[k], ssl)
            scat_start(jbase + k, rows[k], dsl)

    plsc.subcore_barrier()
    pltpu.sync_copy(acc_sh.at[pl.ds(rbase, _ROWS_PT)],
                    out_hbm.at[cid, pl.ds(rbase, _ROWS_PT)])

    @pl.when(sid == _NS - 1)
    def _write_tail():
        tbase = _NS * _ROWS_PT
        pltpu.sync_copy(acc_sh.at[pl.ds(tbase, _ROWS_TAIL)],
                        out_hbm.at[cid, pl.ds(tbase, _ROWS_TAIL)])


def _gather_body(e1_hbm, rel_hbm, h_hbm, remb_hbm, o1_hbm, o2_hbm,
                 idx_v, rows_v, sem):
    wid = lax.axis_index("s") * _NC + lax.axis_index("c")
    base = pl.multiple_of(wid * _BPW, 8)
    pltpu.sync_copy(e1_hbm.at[pl.ds(base, _BPW)], idx_v)
    pltpu.async_copy(h_hbm.at[idx_v], rows_v, sem).wait()
    pltpu.sync_copy(rows_v, o1_hbm.at[pl.ds(base, _BPW)])
    pltpu.sync_copy(rel_hbm.at[pl.ds(base, _BPW)], idx_v)
    pltpu.async_copy(remb_hbm.at[idx_v], rows_v, sem).wait()
    pltpu.sync_copy(rows_v, o2_hbm.at[pl.ds(base, _BPW)])


@functools.lru_cache(maxsize=None)
def _build_segsum_sc():
    return pl.kernel(
        _segsum_body,
        out_type=jax.ShapeDtypeStruct((_NC, _N, _D), jnp.float32),
        mesh=_sc_mesh(),
        scratch_types=(
            [pltpu.VMEM((_EPH,), jnp.int32)] * 2
            + [pltpu.VMEM((_EPH,), jnp.int32)] * 2
            + [pltpu.VMEM((_CHUNK, _D), jnp.float32)] * _NBUF
            + [pltpu.VMEM_SHARED((_N, _D), jnp.float32)]
            + [pltpu.SemaphoreType.DMA] * (_NBUF + 4)
        ),
    )


@functools.lru_cache(maxsize=None)
def _build_gather_sc():
    return pl.kernel(
        _gather_body,
        out_type=(jax.ShapeDtypeStruct((_B, _D), jnp.float32),
                  jax.ShapeDtypeStruct((_B, _D), jnp.float32)),
        mesh=_sc_mesh(),
        scratch_types=[
            pltpu.VMEM((_BPW,), jnp.int32),
            pltpu.VMEM((_BPW, _D), jnp.float32),
            pltpu.SemaphoreType.DMA,
        ],
    )


def _layer_body(h_ref, agg_ref, wm_ref, wa_ref, ua_ref, b_ref, gm_ref, bt_ref,
                out_ref, hpre_ref, stat_ref):
    p = pl.program_id(0)
    i = pl.program_id(1)

    @pl.when(p == 0)
    def _compute():
        h = h_ref[...]
        agg = agg_ref[0] + agg_ref[1]
        a = jnp.dot(agg.astype(jnp.bfloat16), wm_ref[...].astype(jnp.bfloat16),
                    preferred_element_type=jnp.float32)
        ga = jnp.dot(a.astype(jnp.bfloat16), wa_ref[...].astype(jnp.bfloat16),
                     preferred_element_type=jnp.float32) + b_ref[...]
        gh = jnp.dot(h.astype(jnp.bfloat16), ua_ref[...].astype(jnp.bfloat16),
                     preferred_element_type=jnp.float32)
        z = jax.nn.sigmoid(ga[:, :_D] + gh[:, :_D])
        r = jax.nn.sigmoid(ga[:, _D:2 * _D] + gh[:, _D:2 * _D])
        h_til = jnp.tanh(ga[:, 2 * _D:] + r * gh[:, 2 * _D:])
        hp = (1.0 - z) * h + z * h_til
        hpre_ref[pl.ds(i * _BR, _BR), :] = hp

        @pl.when(i == 0)
        def _init():
            stat_ref[...] = jnp.zeros_like(stat_ref)

        stat_ref[0, :] += jnp.sum(hp, axis=0)
        stat_ref[1, :] += jnp.sum(hp * hp, axis=0)

    @pl.when(p == 1)
    def _normalize():
        mu = stat_ref[0, :] * (1.0 / _N)
        var = stat_ref[1, :] * (1.0 / _N) - mu * mu
        hp = hpre_ref[pl.ds(i * _BR, _BR), :]
        xn = (hp - mu) * lax.rsqrt(var + _EPS)
        out_ref[...] = jnp.tanh(gm_ref[...] * xn + bt_ref[...])


_layer_tc = pl.pallas_call(
    _layer_body,
    grid=(2, _NB),
    in_specs=[
        pl.BlockSpec((_BR, _D), lambda p, i: (i, 0)),
        pl.BlockSpec((_NC, _BR, _D), lambda p, i: (0, i, 0)),
        pl.BlockSpec((_D, _D), lambda p, i: (0, 0)),
        pl.BlockSpec((_D, 3 * _D), lambda p, i: (0, 0)),
        pl.BlockSpec((_D, 3 * _D), lambda p, i: (0, 0)),
        pl.BlockSpec((1, 3 * _D), lambda p, i: (0, 0)),
        pl.BlockSpec((1, _D), lambda p, i: (0, 0)),
        pl.BlockSpec((1, _D), lambda p, i: (0, 0)),
    ],
    out_specs=pl.BlockSpec((_BR, _D), lambda p, i: (i, 0)),
    out_shape=jax.ShapeDtypeStruct((_N, _D), jnp.float32),
    scratch_shapes=[
        pltpu.VMEM((_N, _D), jnp.float32),
        pltpu.VMEM((8, _D), jnp.float32),
    ],
)


def _distmult_body(e1_ref, r_ref, h_ref, out_ref):
    q = (e1_ref[...] * r_ref[...]).astype(jnp.bfloat16)
    s = lax.dot_general(h_ref[...].astype(jnp.bfloat16), q,
                        (((1,), (1,)), ((), ())),
                        preferred_element_type=jnp.float32)
    out_ref[...] = jax.nn.sigmoid(s)


_EB = 1000

_distmult_tc = pl.pallas_call(
    _distmult_body,
    grid=(_N // _EB,),
    in_specs=[
        pl.BlockSpec((_B, _D), lambda j: (0, 0)),
        pl.BlockSpec((_B, _D), lambda j: (0, 0)),
        pl.BlockSpec((_EB, _D), lambda j: (j, 0)),
    ],
    out_specs=pl.BlockSpec((_EB, _B), lambda j: (j, 0)),
    out_shape=jax.ShapeDtypeStruct((_N, _B), jnp.float32),
)


def kernel(edge_index, e1, rel, node_table, rel_emb,
           W_msg0, Wa0, Ua0, b0, gamma0, beta0,
           W_msg1, Wa1, Ua1, b1, gamma1, beta1):
    zeros = jnp.zeros((_N, _D), jnp.float32)
    b0r = b0.reshape(1, 3 * _D)
    b1r = b1.reshape(1, 3 * _D)
    g0r = gamma0.reshape(1, _D)
    g1r = gamma1.reshape(1, _D)
    bt0r = beta0.reshape(1, _D)
    bt1r = beta1.reshape(1, _D)

    segsum_sc = _build_segsum_sc()
    gather_sc = _build_gather_sc()
    src = edge_index[0]
    dst = edge_index[1]
    agg0 = segsum_sc(src, dst, node_table, zeros)
    h1 = _layer_tc(node_table, agg0, W_msg0, Wa0, Ua0, b0r, g0r, bt0r)
    agg1 = segsum_sc(src, dst, h1, zeros)
    h2 = _layer_tc(h1, agg1, W_msg1, Wa1, Ua1, b1r, g1r, bt1r)
    e1_emb, r_emb = gather_sc(e1, rel, h2, rel_emb)
    return _distmult_tc(e1_emb, r_emb, h2).T
